# Initial kernel scaffold; baseline (speedup 1.0000x reference)
import jax
import jax.numpy as jnp
from jax.experimental import pallas as pl


def kernel(x, u, w_packed):
    raise NotImplementedError("write your pallas kernel here")


# speedup vs baseline: 1.8200x; 1.8200x over previous
import jax
import jax.numpy as jnp
from jax.experimental import pallas as pl
from jax.experimental.pallas import tpu as pltpu

_TILE_ROWS = 8192


def _readout_body(bias_row, n_out, g_out):
    def body(x_ref, u_ref, w_ref, xo_ref, uo_ref):
        wb = w_ref[0:bias_row, :]
        b = w_ref[bias_row:bias_row + 1, :]

        acc = jnp.dot(x_ref[...], wb, preferred_element_type=jnp.float32)
        xo_ref[...] = jax.nn.sigmoid(acc + b)[:, 0:n_out]

        acc_g = jnp.dot(u_ref[...], wb, preferred_element_type=jnp.float32)
        uo_ref[...] = jax.nn.sigmoid(acc_g + b)[:, n_out:n_out + g_out]

    return body


def kernel(x, u, w_packed):
    n, f_in_n = x.shape
    g, f_in_g = u.shape
    wr, wc = w_packed.shape
    bias_row = wr - 8
    n_out, g_out = 2, 5

    tm = min(_TILE_ROWS, ((n + 7) // 8) * 8)
    grid = (pl.cdiv(n, tm),)

    return pl.pallas_call(
        _readout_body(bias_row, n_out, g_out),
        out_shape=(jax.ShapeDtypeStruct((n, n_out), jnp.float32),
                   jax.ShapeDtypeStruct((g, g_out), jnp.float32)),
        grid=grid,
        in_specs=[pl.BlockSpec((tm, f_in_n), lambda i: (i, 0)),
                  pl.BlockSpec((g, f_in_g), lambda i: (0, 0)),
                  pl.BlockSpec((wr, wc), lambda i: (0, 0))],
        out_specs=(pl.BlockSpec((tm, n_out), lambda i: (i, 0)),
                   pl.BlockSpec((g, g_out), lambda i: (0, 0))),
        compiler_params=pltpu.CompilerParams(
            dimension_semantics=("parallel",)),
    )(x, u, w_packed)


# speedup vs baseline: 1.8441x; 1.0132x over previous
import jax
import jax.numpy as jnp
from jax.experimental import pallas as pl
from jax.experimental.pallas import tpu as pltpu

_TILE_ROWS = 16384


def _readout_body(bias_row, n_out, g_out):
    def body(x_ref, u_ref, w_ref, xo_ref, uo_ref):
        wb = w_ref[0:bias_row, :]
        b = w_ref[bias_row:bias_row + 1, :]

        acc = jnp.dot(x_ref[...], wb, preferred_element_type=jnp.float32)
        xo_ref[...] = jax.nn.sigmoid(acc + b)[:, 0:n_out]

        acc_g = jnp.dot(u_ref[...], wb, preferred_element_type=jnp.float32)
        uo_ref[...] = jax.nn.sigmoid(acc_g + b)[:, n_out:n_out + g_out]

    return body


def kernel(x, u, w_packed):
    n, f_in_n = x.shape
    g, f_in_g = u.shape
    wr, wc = w_packed.shape
    bias_row = wr - 8
    n_out, g_out = 2, 5

    tm = min(_TILE_ROWS, ((n + 7) // 8) * 8)
    grid = (pl.cdiv(n, tm),)

    return pl.pallas_call(
        _readout_body(bias_row, n_out, g_out),
        out_shape=(jax.ShapeDtypeStruct((n, n_out), jnp.float32),
                   jax.ShapeDtypeStruct((g, g_out), jnp.float32)),
        grid=grid,
        in_specs=[pl.BlockSpec((tm, f_in_n), lambda i: (i, 0)),
                  pl.BlockSpec((g, f_in_g), lambda i: (0, 0)),
                  pl.BlockSpec((wr, wc), lambda i: (0, 0))],
        out_specs=(pl.BlockSpec((tm, n_out), lambda i: (i, 0)),
                   pl.BlockSpec((g, g_out), lambda i: (0, 0))),
        compiler_params=pltpu.CompilerParams(
            dimension_semantics=("parallel",)),
    )(x, u, w_packed)
